# baseline (speedup 1.0000x reference)
import jax
import jax.numpy as jnp
from jax.experimental import pallas as pl
from jax.experimental.pallas import tpu as pltpu

_NH = 16
_S = 32
_D = 2 * _S - 1


def _bias_body(tbl_ref, qhbm_ref, out_ref, q_scratch, stage_sem, band_sems):
    h = pl.program_id(0)

    tbl = tbl_ref[0]
    b = jax.lax.broadcasted_iota(jnp.int32, (_D, _S), 0)
    cj = jax.lax.broadcasted_iota(jnp.int32, (_D, _S), 1)
    for ci in range(_S):
        pc = (b == (ci - cj + (_S - 1))).astype(jnp.float32)
        q_scratch[ci] = jnp.dot(tbl, pc, preferred_element_type=jnp.float32)

    stage = pltpu.make_async_copy(q_scratch, qhbm_ref.at[h], stage_sem)
    stage.start()
    stage.wait()

    copies = []
    for ri in range(_S):
        cp = pltpu.make_async_copy(
            qhbm_ref.at[h, :, pl.ds(_S - 1 - ri, _S), :],
            out_ref.at[h, ri],
            band_sems.at[ri],
        )
        cp.start()
        copies.append(cp)
    for cp in copies:
        cp.wait()


def kernel(rel_bias, H, W):
    del H, W
    tbl = jnp.transpose(rel_bias, (2, 0, 1))[:, ::-1, :]
    _, out5 = pl.pallas_call(
        _bias_body,
        grid=(_NH,),
        in_specs=[pl.BlockSpec((1, _D, _D), lambda h: (h, 0, 0))],
        out_specs=[
            pl.BlockSpec(memory_space=pl.MemorySpace.ANY),
            pl.BlockSpec(memory_space=pl.MemorySpace.ANY),
        ],
        out_shape=[
            jax.ShapeDtypeStruct((_NH, _S, _D, _S), jnp.float32),
            jax.ShapeDtypeStruct((_NH, _S, _S, _S, _S), jnp.float32),
        ],
        scratch_shapes=[
            pltpu.VMEM((_S, _D, _S), jnp.float32),
            pltpu.SemaphoreType.DMA,
            pltpu.SemaphoreType.DMA((_S,)),
        ],
    )(tbl)
    return out5.reshape(_NH, _S * _S, _S * _S)


# speedup vs baseline: 55.8870x; 55.8870x over previous
import jax
import jax.numpy as jnp
from jax.experimental import pallas as pl
from jax.experimental.pallas import tpu as pltpu

_NH = 16
_S = 32
_D = 2 * _S - 1
_RB = 4


def _bias_body(tbl_ref, out_ref, r_scratch):
    rg = pl.program_id(1)

    @pl.when(rg == 0)
    def _stage1():
        tbl = tbl_ref[0]
        p = jax.lax.broadcasted_iota(jnp.int32, (64, _D), 0)
        k = jax.lax.broadcasted_iota(jnp.int32, (64, _D), 1)
        ssel = (k == 4 * (p % 16) + p // 16).astype(jnp.float32)
        trs = jnp.dot(ssel, tbl, preferred_element_type=jnp.float32)
        b = jax.lax.broadcasted_iota(jnp.int32, (_D, _S), 0)
        cj = jax.lax.broadcasted_iota(jnp.int32, (_D, _S), 1)
        for ci in range(_S):
            pc = (b == (ci - cj + (_S - 1))).astype(jnp.float32)
            pq = jnp.dot(trs, pc, preferred_element_type=jnp.float32)
            for s in range(4):
                for g in range(4):
                    t = (g + s) % 4
                    r_scratch[s, ci, :, g * _S:(g + 1) * _S] = (
                        pq[t * 16:(t + 1) * 16, :])

    q = 7 - rg
    for r in range(_RB):
        s = 3 - r
        a = r_scratch[s, :, pl.ds(q, 8), :]
        bb = r_scratch[s, :, pl.ds(q + 1, 8), :]
        if s == 0:
            band = a
        else:
            lane = jax.lax.broadcasted_iota(jnp.int32, (_S, 8, 4 * _S), 2)
            band = jnp.where(lane // _S < 4 - s, a, bb)
        out_ref[0, r] = band


def kernel(rel_bias, H, W):
    del H, W
    tbl = jnp.transpose(rel_bias, (2, 0, 1))[:, ::-1, :]
    out5 = pl.pallas_call(
        _bias_body,
        grid=(_NH, _S // _RB),
        in_specs=[pl.BlockSpec((1, _D, _D), lambda h, rg: (h, 0, 0))],
        out_specs=pl.BlockSpec((1, _RB, _S, _S // 4, 4 * _S),
                               lambda h, rg: (h, rg, 0, 0, 0)),
        out_shape=jax.ShapeDtypeStruct((_NH, _S, _S, _S // 4, 4 * _S),
                                       jnp.float32),
        scratch_shapes=[pltpu.VMEM((4, _S, _S // 2, 4 * _S), jnp.float32)],
    )(tbl)
    return out5.reshape(_NH, _S * _S, _S * _S)


# speedup vs baseline: 64.3145x; 1.1508x over previous
import jax
import jax.numpy as jnp
from jax.experimental import pallas as pl
from jax.experimental.pallas import tpu as pltpu

_NH = 16
_S = 32
_D = 2 * _S - 1
_RB = 8


def _bias_body(tbl_ref, out_ref, r_scratch):
    rg = pl.program_id(1)

    @pl.when(rg == 0)
    def _stage1():
        tbl = tbl_ref[0]
        p = jax.lax.broadcasted_iota(jnp.int32, (64, _D), 0)
        k = jax.lax.broadcasted_iota(jnp.int32, (64, _D), 1)
        ssel = (k == 4 * (p % 16) + p // 16).astype(jnp.float32)
        trs = jnp.dot(ssel, tbl, preferred_element_type=jnp.float32)
        b = jax.lax.broadcasted_iota(jnp.int32, (_D, _S), 0)
        cj = jax.lax.broadcasted_iota(jnp.int32, (_D, _S), 1)
        for ci in range(_S):
            pc = (b == (ci - cj + (_S - 1))).astype(jnp.float32)
            pq = jnp.dot(trs, pc, preferred_element_type=jnp.float32)
            for s in range(4):
                for g in range(4):
                    t = (g + s) % 4
                    r_scratch[s, ci, :, g * _S:(g + 1) * _S] = (
                        pq[t * 16:(t + 1) * 16, :])

    q0 = 7 - 2 * rg
    for r in range(_RB):
        s = (3 - r) % 4
        q = q0 - (1 if r >= 4 else 0)
        a = r_scratch[s, :, pl.ds(q, 8), :]
        bb = r_scratch[s, :, pl.ds(q + 1, 8), :]
        if s == 0:
            band = a
        else:
            lane = jax.lax.broadcasted_iota(jnp.int32, (_S, 8, 4 * _S), 2)
            band = jnp.where(lane // _S < 4 - s, a, bb)
        out_ref[0, r] = band


def kernel(rel_bias, H, W):
    del H, W
    tbl = jnp.transpose(rel_bias, (2, 0, 1))[:, ::-1, :]
    out5 = pl.pallas_call(
        _bias_body,
        grid=(_NH, _S // _RB),
        in_specs=[pl.BlockSpec((1, _D, _D), lambda h, rg: (h, 0, 0))],
        out_specs=pl.BlockSpec((1, _RB, _S, _S // 4, 4 * _S),
                               lambda h, rg: (h, rg, 0, 0, 0)),
        out_shape=jax.ShapeDtypeStruct((_NH, _S, _S, _S // 4, 4 * _S),
                                       jnp.float32),
        scratch_shapes=[pltpu.VMEM((4, _S, _S // 2, 4 * _S), jnp.float32)],
    )(tbl)
    return out5.reshape(_NH, _S * _S, _S * _S)


# speedup vs baseline: 72.7312x; 1.1309x over previous
import jax
import jax.numpy as jnp
from jax.experimental import pallas as pl
from jax.experimental.pallas import tpu as pltpu

_NH = 16
_S = 32
_D = 2 * _S - 1
_RB = 16


def _bias_body(tbl_ref, out_ref, r_scratch):
    rg = pl.program_id(1)

    @pl.when(rg == 0)
    def _stage1():
        tbl = tbl_ref[0]
        p = jax.lax.broadcasted_iota(jnp.int32, (64, _D), 0)
        k = jax.lax.broadcasted_iota(jnp.int32, (64, _D), 1)
        ssel = (k == 4 * (p % 16) + p // 16).astype(jnp.float32)
        trs = jnp.dot(ssel, tbl, preferred_element_type=jnp.float32)
        b = jax.lax.broadcasted_iota(jnp.int32, (_D, _S), 0)
        cj = jax.lax.broadcasted_iota(jnp.int32, (_D, _S), 1)
        for ci in range(_S):
            pc = (b == (ci - cj + (_S - 1))).astype(jnp.float32)
            pq = jnp.dot(trs, pc, preferred_element_type=jnp.float32)
            for s in range(4):
                for g in range(4):
                    t = (g + s) % 4
                    r_scratch[s, ci, :, g * _S:(g + 1) * _S] = (
                        pq[t * 16:(t + 1) * 16, :])

    q0 = -(_RB // 4) * rg
    for r in range(_RB):
        s = (3 - r) % 4
        q = (31 - r) // 4 + q0
        a = r_scratch[s, :, pl.ds(q, 8), :]
        bb = r_scratch[s, :, pl.ds(q + 1, 8), :]
        if s == 0:
            band = a
        else:
            lane = jax.lax.broadcasted_iota(jnp.int32, (_S, 8, 4 * _S), 2)
            band = jnp.where(lane // _S < 4 - s, a, bb)
        out_ref[0, r] = band


def kernel(rel_bias, H, W):
    del H, W
    tbl = jnp.transpose(rel_bias, (2, 0, 1))[:, ::-1, :]
    out5 = pl.pallas_call(
        _bias_body,
        grid=(_NH, _S // _RB),
        in_specs=[pl.BlockSpec((1, _D, _D), lambda h, rg: (h, 0, 0))],
        out_specs=pl.BlockSpec((1, _RB, _S, _S // 4, 4 * _S),
                               lambda h, rg: (h, rg, 0, 0, 0)),
        out_shape=jax.ShapeDtypeStruct((_NH, _S, _S, _S // 4, 4 * _S),
                                       jnp.float32),
        scratch_shapes=[pltpu.VMEM((4, _S, _S // 2, 4 * _S), jnp.float32)],
    )(tbl)
    return out5.reshape(_NH, _S * _S, _S * _S)


# speedup vs baseline: 82.7339x; 1.1375x over previous
import jax
import jax.numpy as jnp
from jax.experimental import pallas as pl
from jax.experimental.pallas import tpu as pltpu

_NH = 16
_S = 32
_D = 2 * _S - 1
_RB = 32


def _bias_body(tbl_ref, out_ref, r_scratch):
    rg = pl.program_id(1)

    @pl.when(rg == 0)
    def _stage1():
        tbl = tbl_ref[0]
        p = jax.lax.broadcasted_iota(jnp.int32, (64, _D), 0)
        k = jax.lax.broadcasted_iota(jnp.int32, (64, _D), 1)
        ssel = (k == 4 * (p % 16) + p // 16).astype(jnp.float32)
        trs = jnp.dot(ssel, tbl, preferred_element_type=jnp.float32)
        b = jax.lax.broadcasted_iota(jnp.int32, (_D, _S), 0)
        cj = jax.lax.broadcasted_iota(jnp.int32, (_D, _S), 1)
        for ci in range(_S):
            pc = (b == (ci - cj + (_S - 1))).astype(jnp.float32)
            pq = jnp.dot(trs, pc, preferred_element_type=jnp.float32)
            for s in range(4):
                for g in range(4):
                    t = (g + s) % 4
                    r_scratch[s, ci, :, g * _S:(g + 1) * _S] = (
                        pq[t * 16:(t + 1) * 16, :])

    q0 = -(_RB // 4) * rg
    for r in range(_RB):
        s = (3 - r) % 4
        q = (31 - r) // 4 + q0
        a = r_scratch[s, :, pl.ds(q, 8), :]
        bb = r_scratch[s, :, pl.ds(q + 1, 8), :]
        if s == 0:
            band = a
        else:
            lane = jax.lax.broadcasted_iota(jnp.int32, (_S, 8, 4 * _S), 2)
            band = jnp.where(lane // _S < 4 - s, a, bb)
        out_ref[0, r] = band


def kernel(rel_bias, H, W):
    del H, W
    tbl = jnp.transpose(rel_bias, (2, 0, 1))[:, ::-1, :]
    out5 = pl.pallas_call(
        _bias_body,
        grid=(_NH, _S // _RB),
        in_specs=[pl.BlockSpec((1, _D, _D), lambda h, rg: (h, 0, 0))],
        out_specs=pl.BlockSpec((1, _RB, _S, _S // 4, 4 * _S),
                               lambda h, rg: (h, rg, 0, 0, 0)),
        out_shape=jax.ShapeDtypeStruct((_NH, _S, _S, _S // 4, 4 * _S),
                                       jnp.float32),
        scratch_shapes=[pltpu.VMEM((4, _S, _S // 2, 4 * _S), jnp.float32)],
    )(tbl)
    return out5.reshape(_NH, _S * _S, _S * _S)


# speedup vs baseline: 83.3678x; 1.0077x over previous
import jax
import jax.numpy as jnp
from jax.experimental import pallas as pl
from jax.experimental.pallas import tpu as pltpu

_NH = 16
_S = 32
_D = 2 * _S - 1


def _bias_body(tbl_ref, out_ref, r_scratch):
    tbl = tbl_ref[0]
    p = jax.lax.broadcasted_iota(jnp.int32, (64, _D), 0)
    k = jax.lax.broadcasted_iota(jnp.int32, (64, _D), 1)
    ssel = (k == 4 * (p % 16) + p // 16).astype(jnp.float32)
    trs = jnp.dot(ssel, tbl, preferred_element_type=jnp.float32)
    b = jax.lax.broadcasted_iota(jnp.int32, (_D, _S), 0)
    cj = jax.lax.broadcasted_iota(jnp.int32, (_D, _S), 1)
    for ci in range(_S):
        pc = (b == (ci - cj + (_S - 1))).astype(jnp.float32)
        pq = jnp.dot(trs, pc, preferred_element_type=jnp.float32)
        for s in range(4):
            for g in range(4):
                t = (g + s) % 4
                r_scratch[s, ci, :, g * _S:(g + 1) * _S] = (
                    pq[t * 16:(t + 1) * 16, :])

    for r in range(_S):
        o = _S - 1 - r
        q, s = o // 4, o % 4
        a = r_scratch[s, :, q:q + 8, :]
        if s == 0:
            band = a
        else:
            bb = r_scratch[s, :, q + 1:q + 9, :]
            lane = jax.lax.broadcasted_iota(jnp.int32, (_S, 8, 4 * _S), 2)
            band = jnp.where(lane // _S < 4 - s, a, bb)
        out_ref[0, r] = band


def kernel(rel_bias, H, W):
    del H, W
    tbl = jnp.transpose(rel_bias, (2, 0, 1))[:, ::-1, :]
    out5 = pl.pallas_call(
        _bias_body,
        grid=(_NH,),
        in_specs=[pl.BlockSpec((1, _D, _D), lambda h: (h, 0, 0))],
        out_specs=pl.BlockSpec((1, _S, _S, _S // 4, 4 * _S),
                               lambda h: (h, 0, 0, 0, 0)),
        out_shape=jax.ShapeDtypeStruct((_NH, _S, _S, _S // 4, 4 * _S),
                                       jnp.float32),
        scratch_shapes=[pltpu.VMEM((4, _S, _S // 2, 4 * _S), jnp.float32)],
    )(tbl)
    return out5.reshape(_NH, _S * _S, _S * _S)
